# baseline (speedup 1.0000x reference)
import functools

import jax
import jax.numpy as jnp
from jax import lax
from jax.experimental import pallas as pl
from jax.experimental.pallas import tpu as pltpu
from jax.experimental.pallas import tpu_sc as plsc

_NC = 2
_NS = 16
_NW = _NC * _NS

_B = 4096
_H = 200
_D = 32
_V = 1000000

_HB = 8
_BB = 128
_NBB = _B // _BB
_UPW = (_NBB * (_H // _HB)) // _NW
_C = 256
_NQ = _HB * _BB // _C


_NBLK = _V // 128
_BPW_A = -(-_NBLK // _NW)


@jax.jit
def _sc_transpose(Wt, tail16):
    mesh = plsc.VectorSubcoreMesh(core_axis_name="c", subcore_axis_name="s")

    @functools.partial(
        pl.kernel,
        mesh=mesh,
        out_type=jax.ShapeDtypeStruct((_V // 4, 128), jnp.float32),
        scratch_types=[
            [pltpu.VMEM((32, 128), jnp.float32) for _ in range(2)],
            [pltpu.VMEM((32, 128), jnp.float32) for _ in range(2)],
            [pltpu.SemaphoreType.DMA for _ in range(2)],
            [pltpu.SemaphoreType.DMA for _ in range(2)],
        ],
        compiler_params=pltpu.CompilerParams(needs_layout_passes=False),
    )
    def k(Wt_hbm, tail_hbm, out_hbm, wts, wrs, isems, osems):
        wid = lax.axis_index("s") * _NC + lax.axis_index("c")
        iota = lax.iota(jnp.int32, 16)
        c2base = (iota & 3) * _D

        def fetch(blk, b):
            pltpu.async_copy(
                Wt_hbm.at[:, pl.ds(blk * 128, 128)], wts[b], isems[b])

        def wait_fetch(b):
            pltpu.make_async_copy(
                Wt_hbm.at[:, pl.ds(0, 128)], wts[b], isems[b]).wait()

        def put(blk, b):
            pltpu.async_copy(
                wrs[b], out_hbm.at[pl.ds(blk * 32, 32), :], osems[b])

        def wait_put(b):
            pltpu.make_async_copy(
                wrs[b], out_hbm.at[pl.ds(0, 32), :], osems[b]).wait()

        for b in range(2):
            @pl.when(b * _NW + wid < _NBLK)
            def _():
                fetch(b * _NW + wid, b)

        def body(j, carry):
            for b in range(2):
                i = 2 * j + b
                blk = i * _NW + wid

                @pl.when(blk < _NBLK)
                def _():
                    wait_fetch(b)

                    @pl.when(j >= 1)
                    def _():
                        wait_put(b)

                    @plsc.parallel_loop(0, 8, unroll=4)
                    def _(cg):
                        r2 = cg * 4 + (iota >> 2)
                        for e in range(_D):
                            vals = wts[b][e, pl.ds(cg * 16, 16)]
                            plsc.store_scatter(wrs[b], [r2, c2base + e], vals)

                    put(blk, b)
                    nxt = blk + 2 * _NW

                    @pl.when(nxt < _NBLK)
                    def _():
                        fetch(nxt, b)
            return carry

        lax.fori_loop(0, (_BPW_A + 1) // 2, body, 0, unroll=False)
        for b in range(2):
            wait_put(b)

        @pl.when(wid == _NW - 1)
        def _():
            pltpu.sync_copy(tail_hbm, wts[0].at[pl.ds(0, 16), :])
            pltpu.sync_copy(wts[0].at[pl.ds(0, 16), :],
                            out_hbm.at[pl.ds(_V // 4 - 16, 16), :])

    return k(Wt, tail16)


@jax.jit
def _sc_embed(xT, table2):
    mesh = plsc.VectorSubcoreMesh(core_axis_name="c", subcore_axis_name="s")

    @functools.partial(
        pl.kernel,
        mesh=mesh,
        out_type=jax.ShapeDtypeStruct((_H, _D, _B), jnp.float32),
        scratch_types=[
            pltpu.VMEM((_HB, _BB), jnp.int32),
            [pltpu.VMEM((_C,), jnp.int32) for _ in range(2)],
            [pltpu.VMEM((_C,), jnp.int32) for _ in range(2)],
            [pltpu.VMEM((_C, 128), jnp.float32) for _ in range(2)],
            [pltpu.VMEM((_C // 128, 8, _BB), jnp.float32) for _ in range(8)],
            [pltpu.SemaphoreType.DMA for _ in range(2)],
            [pltpu.SemaphoreType.DMA for _ in range(2)],
        ],
        compiler_params=pltpu.CompilerParams(needs_layout_passes=False),
    )
    def k(xT_hbm, tab_hbm, out_hbm, xt_v, q_v, a_v, rows, ots, gsems, ssems):
        wid = lax.axis_index("s") * _NC + lax.axis_index("c")
        iota = lax.iota(jnp.int32, 16)

        def drain_stores(b):
            for _ in range(8):
                pltpu.make_async_copy(
                    ots[0].at[0],
                    out_hbm.at[0, pl.ds(0, 8), pl.ds(0, _BB)],
                    ssems[b]).wait()

        def unit_body(uu, carry):
            u = uu * _NW + wid
            b0 = (u % _NBB) * _BB
            h0 = (u // _NBB) * _HB
            pltpu.sync_copy(
                xT_hbm.at[pl.ds(h0, _HB), pl.ds(b0, _BB)], xt_v)

            def build_qa(qq, b):
                for g in range(_C // 16):
                    v = xt_v[qq * 2 + g // 8, pl.ds((g % 8) * 16, 16)]
                    q_v[b][pl.ds(g * 16, 16)] = v >> 2
                    a_v[b][pl.ds(g * 16, 16)] = v & 3

            def gather_q(b):
                return pltpu.async_copy(
                    tab_hbm.at[q_v[b]], rows[b], gsems[b])

            build_qa(0, 0)
            build_qa(1, 1)
            gathers = [gather_q(0), gather_q(1)]
            for qq in range(_NQ):
                b = qq % 2
                gathers[qq].wait()
                if qq < 2:
                    @pl.when(uu != 0)
                    def _():
                        drain_stores(b)
                else:
                    drain_stores(b)

                @plsc.parallel_loop(0, _C // 16, unroll=4)
                def _(j16):
                    ridx = j16 * 16 + iota
                    a16 = a_v[b][pl.ds(j16 * 16, 16)]
                    cbase = a16 * _D
                    hh = j16 >> 3
                    cg16 = (j16 & 7) * 16
                    for e in range(_D):
                        vals = plsc.load_gather(rows[b], [ridx, cbase + e])
                        ots[b * 4 + (e >> 3)][hh, e & 7,
                                              pl.ds(cg16, 16)] = vals

                for hh in range(2):
                    for t0 in range(4):
                        pltpu.async_copy(
                            ots[b * 4 + t0].at[hh],
                            out_hbm.at[h0 + qq * 2 + hh, pl.ds(t0 * 8, 8),
                                       pl.ds(b0, _BB)],
                            ssems[b])
                nxt = qq + 2
                if nxt < _NQ:
                    build_qa(nxt, b)
                    gathers.append(gather_q(b))
            return carry

        lax.fori_loop(0, _UPW, unit_body, 0, unroll=False)
        for b in range(2):
            drain_stores(b)

    return k(xT, table2)


def kernel(x, W):
    xT = x.T.astype(jnp.int32)
    tail16 = lax.slice(W, (_V - 64, 0), (_V, _D)).reshape(16, 128)
    table2 = _sc_transpose(W.T, tail16)
    outT = _sc_embed(xT, table2)
    return outT.transpose(2, 0, 1)


# speedup vs baseline: 1.1194x; 1.1194x over previous
import functools

import jax
import jax.numpy as jnp
from jax import lax
from jax.experimental import pallas as pl
from jax.experimental.pallas import tpu as pltpu
from jax.experimental.pallas import tpu_sc as plsc

_NC = 2
_NS = 16
_NW = _NC * _NS

_B = 4096
_H = 200
_D = 32
_V = 1000000

_HB = 8
_BB = 128
_NBB = _B // _BB
_UPW = (_NBB * (_H // _HB)) // _NW
_C = 256
_NQ = _HB * _BB // _C


_NBLK = _V // 256
_BPW_A = -(-_NBLK // _NW)


@jax.jit
def _sc_transpose(Wt, tail16):
    mesh = plsc.VectorSubcoreMesh(core_axis_name="c", subcore_axis_name="s")

    @functools.partial(
        pl.kernel,
        mesh=mesh,
        out_type=jax.ShapeDtypeStruct((_V // 4, 128), jnp.float32),
        scratch_types=[
            [pltpu.VMEM((32, 256), jnp.float32) for _ in range(2)],
            [pltpu.VMEM((64, 128), jnp.float32) for _ in range(2)],
            [pltpu.SemaphoreType.DMA for _ in range(2)],
            [pltpu.SemaphoreType.DMA for _ in range(2)],
        ],
        compiler_params=pltpu.CompilerParams(needs_layout_passes=False),
    )
    def k(Wt_hbm, tail_hbm, out_hbm, wts, wrs, isems, osems):
        wid = lax.axis_index("s") * _NC + lax.axis_index("c")
        iota = lax.iota(jnp.int32, 16)
        c2base = (iota & 3) * _D

        def fetch(blk, b):
            pltpu.async_copy(
                Wt_hbm.at[:, pl.ds(blk * 256, 256)], wts[b], isems[b])

        def wait_fetch(b):
            pltpu.make_async_copy(
                Wt_hbm.at[:, pl.ds(0, 256)], wts[b], isems[b]).wait()

        def put(blk, b):
            pltpu.async_copy(
                wrs[b], out_hbm.at[pl.ds(blk * 64, 64), :], osems[b])

        def wait_put(b):
            pltpu.make_async_copy(
                wrs[b], out_hbm.at[pl.ds(0, 64), :], osems[b]).wait()

        for b in range(2):
            @pl.when(b * _NW + wid < _NBLK)
            def _():
                fetch(b * _NW + wid, b)

        def body(j, carry):
            for b in range(2):
                i = 2 * j + b
                blk = i * _NW + wid

                @pl.when(blk < _NBLK)
                def _():
                    wait_fetch(b)

                    @pl.when(j >= 1)
                    def _():
                        wait_put(b)

                    @plsc.parallel_loop(0, 16, unroll=2)
                    def _(cg):
                        r2 = cg * 4 + (iota >> 2)
                        for e in range(_D):
                            vals = wts[b][e, pl.ds(cg * 16, 16)]
                            plsc.store_scatter(wrs[b], [r2, c2base + e], vals)

                    put(blk, b)
                    nxt = blk + 2 * _NW

                    @pl.when(nxt < _NBLK)
                    def _():
                        fetch(nxt, b)
            return carry

        lax.fori_loop(0, (_BPW_A + 1) // 2, body, 0, unroll=False)
        for b in range(2):
            wait_put(b)

        @pl.when(wid == _NW - 1)
        def _():
            pltpu.sync_copy(tail_hbm, wts[0].at[pl.ds(0, 16), pl.ds(0, 128)])
            pltpu.sync_copy(wts[0].at[pl.ds(0, 16), pl.ds(0, 128)],
                            out_hbm.at[pl.ds(_V // 4 - 16, 16), :])

    return k(Wt, tail16)


@jax.jit
def _sc_embed(xT, table2):
    mesh = plsc.VectorSubcoreMesh(core_axis_name="c", subcore_axis_name="s")

    @functools.partial(
        pl.kernel,
        mesh=mesh,
        out_type=jax.ShapeDtypeStruct((_H, _D, _B), jnp.float32),
        scratch_types=[
            pltpu.VMEM((_HB, _BB), jnp.int32),
            [pltpu.VMEM((_C,), jnp.int32) for _ in range(2)],
            [pltpu.VMEM((_C,), jnp.int32) for _ in range(2)],
            [pltpu.VMEM((_C, 128), jnp.float32) for _ in range(2)],
            [pltpu.VMEM((_C // 128, 8, _BB), jnp.float32) for _ in range(8)],
            [pltpu.SemaphoreType.DMA for _ in range(2)],
            [pltpu.SemaphoreType.DMA for _ in range(2)],
        ],
        compiler_params=pltpu.CompilerParams(needs_layout_passes=False),
    )
    def k(xT_hbm, tab_hbm, out_hbm, xt_v, q_v, a_v, rows, ots, gsems, ssems):
        wid = lax.axis_index("s") * _NC + lax.axis_index("c")
        iota = lax.iota(jnp.int32, 16)

        def drain_stores(b):
            for _ in range(8):
                pltpu.make_async_copy(
                    ots[0].at[0],
                    out_hbm.at[0, pl.ds(0, 8), pl.ds(0, _BB)],
                    ssems[b]).wait()

        def unit_body(uu, carry):
            u = uu * _NW + wid
            b0 = (u % _NBB) * _BB
            h0 = (u // _NBB) * _HB
            pltpu.sync_copy(
                xT_hbm.at[pl.ds(h0, _HB), pl.ds(b0, _BB)], xt_v)

            def build_qa(qq, b):
                for g in range(_C // 16):
                    v = xt_v[qq * 2 + g // 8, pl.ds((g % 8) * 16, 16)]
                    q_v[b][pl.ds(g * 16, 16)] = v >> 2
                    a_v[b][pl.ds(g * 16, 16)] = v & 3

            def gather_q(b):
                return pltpu.async_copy(
                    tab_hbm.at[q_v[b]], rows[b], gsems[b])

            build_qa(0, 0)
            build_qa(1, 1)
            gathers = [gather_q(0), gather_q(1)]
            for qq in range(_NQ):
                b = qq % 2
                gathers[qq].wait()
                if qq < 2:
                    @pl.when(uu != 0)
                    def _():
                        drain_stores(b)
                else:
                    drain_stores(b)

                @plsc.parallel_loop(0, _C // 16, unroll=2)
                def _(j16):
                    ridx = j16 * 16 + iota
                    a16 = a_v[b][pl.ds(j16 * 16, 16)]
                    cbase = a16 * _D
                    hh = j16 >> 3
                    cg16 = (j16 & 7) * 16
                    for e in range(_D):
                        vals = plsc.load_gather(rows[b], [ridx, cbase + e])
                        ots[b * 4 + (e >> 3)][hh, e & 7,
                                              pl.ds(cg16, 16)] = vals

                for hh in range(2):
                    for t0 in range(4):
                        pltpu.async_copy(
                            ots[b * 4 + t0].at[hh],
                            out_hbm.at[h0 + qq * 2 + hh, pl.ds(t0 * 8, 8),
                                       pl.ds(b0, _BB)],
                            ssems[b])
                nxt = qq + 2
                if nxt < _NQ:
                    build_qa(nxt, b)
                    gathers.append(gather_q(b))
            return carry

        lax.fori_loop(0, _UPW, unit_body, 0, unroll=False)
        for b in range(2):
            drain_stores(b)

    return k(xT, table2)


def kernel(x, W):
    xT = x.T.astype(jnp.int32)
    tail16 = lax.slice(W, (_V - 64, 0), (_V, _D)).reshape(16, 128)
    table2 = _sc_transpose(W.T, tail16)
    outT = _sc_embed(xT, table2)
    return outT.transpose(2, 0, 1)


# speedup vs baseline: 1.1277x; 1.0075x over previous
import functools

import jax
import jax.numpy as jnp
from jax import lax
from jax.experimental import pallas as pl
from jax.experimental.pallas import tpu as pltpu
from jax.experimental.pallas import tpu_sc as plsc

_NC = 2
_NS = 16
_NW = _NC * _NS

_B = 4096
_H = 200
_D = 32
_V = 1000000

_HB = 8
_BB = 128
_NBB = _B // _BB
_UPW = (_NBB * (_H // _HB)) // _NW
_C = 256
_NQ = _HB * _BB // _C


_NBLK = _V // 256
_BPW_A = -(-_NBLK // _NW)


@jax.jit
def _sc_transpose(Wt, tail16):
    mesh = plsc.VectorSubcoreMesh(core_axis_name="c", subcore_axis_name="s")

    @functools.partial(
        pl.kernel,
        mesh=mesh,
        out_type=jax.ShapeDtypeStruct((_V // 4, 128), jnp.float32),
        scratch_types=[
            [pltpu.VMEM((32, 256), jnp.float32) for _ in range(2)],
            [pltpu.VMEM((64, 128), jnp.float32) for _ in range(2)],
            [pltpu.SemaphoreType.DMA for _ in range(2)],
            [pltpu.SemaphoreType.DMA for _ in range(2)],
        ],
        compiler_params=pltpu.CompilerParams(needs_layout_passes=False),
    )
    def k(Wt_hbm, tail_hbm, out_hbm, wts, wrs, isems, osems):
        wid = lax.axis_index("s") * _NC + lax.axis_index("c")
        iota = lax.iota(jnp.int32, 16)
        c2base = (iota & 3) * _D

        def fetch(blk, b):
            pltpu.async_copy(
                Wt_hbm.at[:, pl.ds(blk * 256, 256)], wts[b], isems[b])

        def wait_fetch(b):
            pltpu.make_async_copy(
                Wt_hbm.at[:, pl.ds(0, 256)], wts[b], isems[b]).wait()

        def put(blk, b):
            pltpu.async_copy(
                wrs[b], out_hbm.at[pl.ds(blk * 64, 64), :], osems[b])

        def wait_put(b):
            pltpu.make_async_copy(
                wrs[b], out_hbm.at[pl.ds(0, 64), :], osems[b]).wait()

        for b in range(2):
            @pl.when(b * _NW + wid < _NBLK)
            def _():
                fetch(b * _NW + wid, b)

        def body(j, carry):
            for b in range(2):
                i = 2 * j + b
                blk = i * _NW + wid

                @pl.when(blk < _NBLK)
                def _():
                    wait_fetch(b)

                    @pl.when(j >= 1)
                    def _():
                        wait_put(b)

                    @plsc.parallel_loop(0, 16, unroll=2)
                    def _(cg):
                        r2 = cg * 4 + (iota >> 2)
                        vals = [wts[b][e, pl.ds(cg * 16, 16)]
                                for e in range(_D)]
                        for e in range(_D):
                            plsc.store_scatter(
                                wrs[b], [r2, c2base + e], vals[e])

                    put(blk, b)
                    nxt = blk + 2 * _NW

                    @pl.when(nxt < _NBLK)
                    def _():
                        fetch(nxt, b)
            return carry

        lax.fori_loop(0, (_BPW_A + 1) // 2, body, 0, unroll=False)
        for b in range(2):
            wait_put(b)

        @pl.when(wid == _NW - 1)
        def _():
            pltpu.sync_copy(tail_hbm, wts[0].at[pl.ds(0, 16), pl.ds(0, 128)])
            pltpu.sync_copy(wts[0].at[pl.ds(0, 16), pl.ds(0, 128)],
                            out_hbm.at[pl.ds(_V // 4 - 16, 16), :])

    return k(Wt, tail16)


@jax.jit
def _sc_embed(xT, table2):
    mesh = plsc.VectorSubcoreMesh(core_axis_name="c", subcore_axis_name="s")

    @functools.partial(
        pl.kernel,
        mesh=mesh,
        out_type=jax.ShapeDtypeStruct((_H, _D, _B), jnp.float32),
        scratch_types=[
            pltpu.VMEM((_HB, _BB), jnp.int32),
            [pltpu.VMEM((_C,), jnp.int32) for _ in range(2)],
            [pltpu.VMEM((_C,), jnp.int32) for _ in range(2)],
            [pltpu.VMEM((_C, 128), jnp.float32) for _ in range(2)],
            [pltpu.VMEM((_C // 128, 8, _BB), jnp.float32) for _ in range(8)],
            [pltpu.SemaphoreType.DMA for _ in range(2)],
            [pltpu.SemaphoreType.DMA for _ in range(2)],
        ],
        compiler_params=pltpu.CompilerParams(needs_layout_passes=False),
    )
    def k(xT_hbm, tab_hbm, out_hbm, xt_v, q_v, a_v, rows, ots, gsems, ssems):
        wid = lax.axis_index("s") * _NC + lax.axis_index("c")
        iota = lax.iota(jnp.int32, 16)

        def drain_stores(b):
            for _ in range(8):
                pltpu.make_async_copy(
                    ots[0].at[0],
                    out_hbm.at[0, pl.ds(0, 8), pl.ds(0, _BB)],
                    ssems[b]).wait()

        def unit_body(uu, carry):
            u = uu * _NW + wid
            b0 = (u % _NBB) * _BB
            h0 = (u // _NBB) * _HB
            pltpu.sync_copy(
                xT_hbm.at[pl.ds(h0, _HB), pl.ds(b0, _BB)], xt_v)

            def build_qa(qq, b):
                for g in range(_C // 16):
                    v = xt_v[qq * 2 + g // 8, pl.ds((g % 8) * 16, 16)]
                    q_v[b][pl.ds(g * 16, 16)] = v >> 2
                    a_v[b][pl.ds(g * 16, 16)] = v & 3

            def gather_q(b):
                return pltpu.async_copy(
                    tab_hbm.at[q_v[b]], rows[b], gsems[b])

            build_qa(0, 0)
            build_qa(1, 1)
            gathers = [gather_q(0), gather_q(1)]
            for qq in range(_NQ):
                b = qq % 2
                gathers[qq].wait()
                if qq < 2:
                    @pl.when(uu != 0)
                    def _():
                        drain_stores(b)
                else:
                    drain_stores(b)

                @plsc.parallel_loop(0, _C // 16, unroll=2)
                def _(j16):
                    ridx = j16 * 16 + iota
                    a16 = a_v[b][pl.ds(j16 * 16, 16)]
                    cbase = a16 * _D
                    hh = j16 >> 3
                    cg16 = (j16 & 7) * 16
                    vals = [plsc.load_gather(rows[b], [ridx, cbase + e])
                            for e in range(_D)]
                    for e in range(_D):
                        ots[b * 4 + (e >> 3)][hh, e & 7,
                                              pl.ds(cg16, 16)] = vals[e]

                for hh in range(2):
                    for t0 in range(4):
                        pltpu.async_copy(
                            ots[b * 4 + t0].at[hh],
                            out_hbm.at[h0 + qq * 2 + hh, pl.ds(t0 * 8, 8),
                                       pl.ds(b0, _BB)],
                            ssems[b])
                nxt = qq + 2
                if nxt < _NQ:
                    build_qa(nxt, b)
                    gathers.append(gather_q(b))
            return carry

        lax.fori_loop(0, _UPW, unit_body, 0, unroll=False)
        for b in range(2):
            drain_stores(b)

    return k(xT, table2)


def kernel(x, W):
    xT = x.T.astype(jnp.int32)
    tail16 = lax.slice(W, (_V - 64, 0), (_V, _D)).reshape(16, 128)
    table2 = _sc_transpose(W.T, tail16)
    outT = _sc_embed(xT, table2)
    return outT.transpose(2, 0, 1)


# speedup vs baseline: 1.1383x; 1.0093x over previous
import functools

import jax
import jax.numpy as jnp
from jax import lax
from jax.experimental import pallas as pl
from jax.experimental.pallas import tpu as pltpu
from jax.experimental.pallas import tpu_sc as plsc

_NC = 2
_NS = 16
_NW = _NC * _NS

_B = 4096
_H = 200
_D = 32
_V = 1000000

_HB = 8
_BB = 128
_NBB = _B // _BB
_UPW = (_NBB * (_H // _HB)) // _NW
_C = 256
_NQ = _HB * _BB // _C


_NBLK = _V // 256
_BPW_A = -(-_NBLK // _NW)


@jax.jit
def _sc_transpose(Wt, tail16):
    mesh = plsc.VectorSubcoreMesh(core_axis_name="c", subcore_axis_name="s")

    @functools.partial(
        pl.kernel,
        mesh=mesh,
        out_type=jax.ShapeDtypeStruct((_V // 4, 128), jnp.float32),
        scratch_types=[
            [pltpu.VMEM((32, 256), jnp.float32) for _ in range(2)],
            [pltpu.VMEM((64, 128), jnp.float32) for _ in range(2)],
            [pltpu.SemaphoreType.DMA for _ in range(2)],
            [pltpu.SemaphoreType.DMA for _ in range(2)],
        ],
        compiler_params=pltpu.CompilerParams(needs_layout_passes=False),
    )
    def k(Wt_hbm, tail_hbm, out_hbm, wts, wrs, isems, osems):
        wid = lax.axis_index("s") * _NC + lax.axis_index("c")
        iota = lax.iota(jnp.int32, 16)
        c2base = (iota & 3) * _D

        def fetch(blk, b):
            pltpu.async_copy(
                Wt_hbm.at[:, pl.ds(blk * 256, 256)], wts[b], isems[b])

        def wait_fetch(b):
            pltpu.make_async_copy(
                Wt_hbm.at[:, pl.ds(0, 256)], wts[b], isems[b]).wait()

        def put(blk, b):
            pltpu.async_copy(
                wrs[b], out_hbm.at[pl.ds(blk * 64, 64), :], osems[b])

        def wait_put(b):
            pltpu.make_async_copy(
                wrs[b], out_hbm.at[pl.ds(0, 64), :], osems[b]).wait()

        for b in range(2):
            @pl.when(b * _NW + wid < _NBLK)
            def _():
                fetch(b * _NW + wid, b)

        def body(j, carry):
            for b in range(2):
                i = 2 * j + b
                blk = i * _NW + wid

                @pl.when(blk < _NBLK)
                def _():
                    wait_fetch(b)

                    @pl.when(j >= 1)
                    def _():
                        wait_put(b)

                    @plsc.parallel_loop(0, 16, unroll=4)
                    def _(cg):
                        r2 = cg * 4 + (iota >> 2)
                        vals = [wts[b][e, pl.ds(cg * 16, 16)]
                                for e in range(_D)]
                        for e in range(_D):
                            plsc.store_scatter(
                                wrs[b], [r2, c2base + e], vals[e])

                    put(blk, b)
                    nxt = blk + 2 * _NW

                    @pl.when(nxt < _NBLK)
                    def _():
                        fetch(nxt, b)
            return carry

        lax.fori_loop(0, (_BPW_A + 1) // 2, body, 0, unroll=False)
        for b in range(2):
            wait_put(b)

        @pl.when(wid == _NW - 1)
        def _():
            pltpu.sync_copy(tail_hbm, wts[0].at[pl.ds(0, 16), pl.ds(0, 128)])
            pltpu.sync_copy(wts[0].at[pl.ds(0, 16), pl.ds(0, 128)],
                            out_hbm.at[pl.ds(_V // 4 - 16, 16), :])

    return k(Wt, tail16)


@jax.jit
def _sc_embed(xT, table2):
    mesh = plsc.VectorSubcoreMesh(core_axis_name="c", subcore_axis_name="s")

    @functools.partial(
        pl.kernel,
        mesh=mesh,
        out_type=jax.ShapeDtypeStruct((_H, _D, _B), jnp.float32),
        scratch_types=[
            pltpu.VMEM((_HB, _BB), jnp.int32),
            [pltpu.VMEM((_C,), jnp.int32) for _ in range(2)],
            [pltpu.VMEM((_C,), jnp.int32) for _ in range(2)],
            [pltpu.VMEM((_C, 128), jnp.float32) for _ in range(2)],
            [pltpu.VMEM((_C // 128, 8, _BB), jnp.float32) for _ in range(8)],
            [pltpu.SemaphoreType.DMA for _ in range(2)],
            [pltpu.SemaphoreType.DMA for _ in range(2)],
        ],
        compiler_params=pltpu.CompilerParams(needs_layout_passes=False),
    )
    def k(xT_hbm, tab_hbm, out_hbm, xt_v, q_v, a_v, rows, ots, gsems, ssems):
        wid = lax.axis_index("s") * _NC + lax.axis_index("c")
        iota = lax.iota(jnp.int32, 16)

        def drain_stores(b):
            for _ in range(8):
                pltpu.make_async_copy(
                    ots[0].at[0],
                    out_hbm.at[0, pl.ds(0, 8), pl.ds(0, _BB)],
                    ssems[b]).wait()

        def unit_body(uu, carry):
            u = uu * _NW + wid
            b0 = (u % _NBB) * _BB
            h0 = (u // _NBB) * _HB
            pltpu.sync_copy(
                xT_hbm.at[pl.ds(h0, _HB), pl.ds(b0, _BB)], xt_v)

            def build_qa(qq, b):
                for g in range(_C // 16):
                    v = xt_v[qq * 2 + g // 8, pl.ds((g % 8) * 16, 16)]
                    q_v[b][pl.ds(g * 16, 16)] = v >> 2
                    a_v[b][pl.ds(g * 16, 16)] = v & 3

            def gather_q(b):
                return pltpu.async_copy(
                    tab_hbm.at[q_v[b]], rows[b], gsems[b])

            build_qa(0, 0)
            build_qa(1, 1)
            gathers = [gather_q(0), gather_q(1)]
            for qq in range(_NQ):
                b = qq % 2
                gathers[qq].wait()
                if qq < 2:
                    @pl.when(uu != 0)
                    def _():
                        drain_stores(b)
                else:
                    drain_stores(b)

                @plsc.parallel_loop(0, _C // 16, unroll=2)
                def _(j16):
                    ridx = j16 * 16 + iota
                    a16 = a_v[b][pl.ds(j16 * 16, 16)]
                    cbase = a16 * _D
                    hh = j16 >> 3
                    cg16 = (j16 & 7) * 16
                    vals = [plsc.load_gather(rows[b], [ridx, cbase + e])
                            for e in range(_D)]
                    for e in range(_D):
                        ots[b * 4 + (e >> 3)][hh, e & 7,
                                              pl.ds(cg16, 16)] = vals[e]

                for hh in range(2):
                    for t0 in range(4):
                        pltpu.async_copy(
                            ots[b * 4 + t0].at[hh],
                            out_hbm.at[h0 + qq * 2 + hh, pl.ds(t0 * 8, 8),
                                       pl.ds(b0, _BB)],
                            ssems[b])
                nxt = qq + 2
                if nxt < _NQ:
                    build_qa(nxt, b)
                    gathers.append(gather_q(b))
            return carry

        lax.fori_loop(0, _UPW, unit_body, 0, unroll=False)
        for b in range(2):
            drain_stores(b)

    return k(xT, table2)


def kernel(x, W):
    xT = x.T.astype(jnp.int32)
    tail16 = lax.slice(W, (_V - 64, 0), (_V, _D)).reshape(16, 128)
    table2 = _sc_transpose(W.T, tail16)
    outT = _sc_embed(xT, table2)
    return outT.transpose(2, 0, 1)


# speedup vs baseline: 1.1599x; 1.0190x over previous
import functools

import jax
import jax.numpy as jnp
from jax import lax
from jax.experimental import pallas as pl
from jax.experimental.pallas import tpu as pltpu
from jax.experimental.pallas import tpu_sc as plsc

_NC = 2
_NS = 16
_NW = _NC * _NS

_B = 4096
_H = 200
_D = 32
_V = 1000000

_HB = 8
_BB = 128
_NBB = _B // _BB
_UPW = (_NBB * (_H // _HB)) // _NW
_C = 256
_NQ = _HB * _BB // _C


_NBLK = _V // 256
_BPW_A = -(-_NBLK // _NW)


@jax.jit
def _sc_transpose(Wt, tail16):
    mesh = plsc.VectorSubcoreMesh(core_axis_name="c", subcore_axis_name="s")

    @functools.partial(
        pl.kernel,
        mesh=mesh,
        out_type=jax.ShapeDtypeStruct((_V // 4, 128), jnp.float32),
        scratch_types=[
            [pltpu.VMEM((32, 256), jnp.float32) for _ in range(2)],
            [pltpu.VMEM((64, 128), jnp.float32) for _ in range(2)],
            [pltpu.SemaphoreType.DMA for _ in range(2)],
            [pltpu.SemaphoreType.DMA for _ in range(2)],
        ],
        compiler_params=pltpu.CompilerParams(needs_layout_passes=False),
    )
    def k(Wt_hbm, tail_hbm, out_hbm, wts, wrs, isems, osems):
        wid = lax.axis_index("s") * _NC + lax.axis_index("c")
        iota = lax.iota(jnp.int32, 16)
        c2base = (iota & 3) * _D

        def fetch(blk, b):
            pltpu.async_copy(
                Wt_hbm.at[:, pl.ds(blk * 256, 256)], wts[b], isems[b])

        def wait_fetch(b):
            pltpu.make_async_copy(
                Wt_hbm.at[:, pl.ds(0, 256)], wts[b], isems[b]).wait()

        def put(blk, b):
            pltpu.async_copy(
                wrs[b], out_hbm.at[pl.ds(blk * 64, 64), :], osems[b])

        def wait_put(b):
            pltpu.make_async_copy(
                wrs[b], out_hbm.at[pl.ds(0, 64), :], osems[b]).wait()

        for b in range(2):
            @pl.when(b * _NW + wid < _NBLK)
            def _():
                fetch(b * _NW + wid, b)

        def body(j, carry):
            for b in range(2):
                i = 2 * j + b
                blk = i * _NW + wid

                @pl.when(blk < _NBLK)
                def _():
                    wait_fetch(b)

                    @pl.when(j >= 1)
                    def _():
                        wait_put(b)

                    @plsc.parallel_loop(0, 16, unroll=4)
                    def _(cg):
                        r2 = cg * 4 + (iota >> 2)
                        vals = [wts[b][e, pl.ds(cg * 16, 16)]
                                for e in range(_D)]
                        for e in range(_D):
                            plsc.store_scatter(
                                wrs[b], [r2, c2base + e], vals[e])

                    put(blk, b)
                    nxt = blk + 2 * _NW

                    @pl.when(nxt < _NBLK)
                    def _():
                        fetch(nxt, b)
            return carry

        lax.fori_loop(0, (_BPW_A + 1) // 2, body, 0, unroll=False)
        for b in range(2):
            wait_put(b)

        @pl.when(wid == _NW - 1)
        def _():
            pltpu.sync_copy(tail_hbm, wts[0].at[pl.ds(0, 16), pl.ds(0, 128)])
            pltpu.sync_copy(wts[0].at[pl.ds(0, 16), pl.ds(0, 128)],
                            out_hbm.at[pl.ds(_V // 4 - 16, 16), :])

    return k(Wt, tail16)


@jax.jit
def _sc_embed(xT, table2):
    mesh = plsc.VectorSubcoreMesh(core_axis_name="c", subcore_axis_name="s")

    @functools.partial(
        pl.kernel,
        mesh=mesh,
        out_type=jax.ShapeDtypeStruct((_H, _D, _B), jnp.float32),
        scratch_types=[
            pltpu.VMEM((_HB, _BB), jnp.int32),
            [pltpu.VMEM((_C,), jnp.int32) for _ in range(2)],
            [pltpu.VMEM((_C,), jnp.int32) for _ in range(2)],
            [pltpu.VMEM((_C, 128), jnp.float32) for _ in range(2)],
            [pltpu.VMEM((_C // 128, 8, _BB), jnp.float32) for _ in range(8)],
            [pltpu.SemaphoreType.DMA for _ in range(2)],
            [pltpu.SemaphoreType.DMA for _ in range(2)],
        ],
        compiler_params=pltpu.CompilerParams(needs_layout_passes=False),
    )
    def k(xT_hbm, tab_hbm, out_hbm, xt_v, q_v, a_v, rows, ots, gsems, ssems):
        wid = lax.axis_index("s") * _NC + lax.axis_index("c")
        iota = lax.iota(jnp.int32, 16)

        def drain_stores(b):
            for _ in range(8):
                pltpu.make_async_copy(
                    ots[0].at[0],
                    out_hbm.at[0, pl.ds(0, 8), pl.ds(0, _BB)],
                    ssems[b]).wait()

        def unit_body(uu, carry):
            u = uu * _NW + wid
            b0 = (u % _NBB) * _BB
            h0 = (u // _NBB) * _HB
            pltpu.sync_copy(
                xT_hbm.at[pl.ds(h0, _HB), pl.ds(b0, _BB)], xt_v)

            def build_qa(qq, b):
                for g in range(_C // 16):
                    v = xt_v[qq * 2 + g // 8, pl.ds((g % 8) * 16, 16)]
                    q_v[b][pl.ds(g * 16, 16)] = v >> 2
                    a_v[b][pl.ds(g * 16, 16)] = v & 3

            def gather_q(b):
                return pltpu.async_copy(
                    tab_hbm.at[q_v[b]], rows[b], gsems[b])

            build_qa(0, 0)
            build_qa(1, 1)
            gathers = [gather_q(0), gather_q(1)]
            for qq in range(_NQ):
                b = qq % 2
                gathers[qq].wait()
                if qq < 2:
                    @pl.when(uu != 0)
                    def _():
                        drain_stores(b)
                else:
                    drain_stores(b)

                @plsc.parallel_loop(0, _C // 16, unroll=4)
                def _(j16):
                    ridx = j16 * 16 + iota
                    a16 = a_v[b][pl.ds(j16 * 16, 16)]
                    cbase = a16 * _D
                    hh = j16 >> 3
                    cg16 = (j16 & 7) * 16
                    vals = [plsc.load_gather(rows[b], [ridx, cbase + e])
                            for e in range(_D)]
                    for e in range(_D):
                        ots[b * 4 + (e >> 3)][hh, e & 7,
                                              pl.ds(cg16, 16)] = vals[e]

                for hh in range(2):
                    for t0 in range(4):
                        pltpu.async_copy(
                            ots[b * 4 + t0].at[hh],
                            out_hbm.at[h0 + qq * 2 + hh, pl.ds(t0 * 8, 8),
                                       pl.ds(b0, _BB)],
                            ssems[b])
                nxt = qq + 2
                if nxt < _NQ:
                    build_qa(nxt, b)
                    gathers.append(gather_q(b))
            return carry

        lax.fori_loop(0, _UPW, unit_body, 0, unroll=False)
        for b in range(2):
            drain_stores(b)

    return k(xT, table2)


def kernel(x, W):
    xT = x.T.astype(jnp.int32)
    tail16 = lax.slice(W, (_V - 64, 0), (_V, _D)).reshape(16, 128)
    table2 = _sc_transpose(W.T, tail16)
    outT = _sc_embed(xT, table2)
    return outT.transpose(2, 0, 1)


# speedup vs baseline: 1.1711x; 1.0096x over previous
import functools

import jax
import jax.numpy as jnp
from jax import lax
from jax.experimental import pallas as pl
from jax.experimental.pallas import tpu as pltpu
from jax.experimental.pallas import tpu_sc as plsc

_NC = 2
_NS = 16
_NW = _NC * _NS

_B = 4096
_H = 200
_D = 32
_V = 1000000

_HB = 8
_BB = 128
_NBB = _B // _BB
_UPW = (_NBB * (_H // _HB)) // _NW
_C = 256
_NQ = _HB * _BB // _C


_NBLK = _V // 256
_BPW_A = -(-_NBLK // _NW)


@jax.jit
def _sc_transpose(Wt, tail16):
    mesh = plsc.VectorSubcoreMesh(core_axis_name="c", subcore_axis_name="s")

    @functools.partial(
        pl.kernel,
        mesh=mesh,
        out_type=jax.ShapeDtypeStruct((_V // 4, 128), jnp.float32),
        scratch_types=[
            [pltpu.VMEM((32, 256), jnp.float32) for _ in range(2)],
            [pltpu.VMEM((64, 128), jnp.float32) for _ in range(2)],
            [pltpu.SemaphoreType.DMA for _ in range(2)],
            [pltpu.SemaphoreType.DMA for _ in range(2)],
        ],
        compiler_params=pltpu.CompilerParams(needs_layout_passes=False),
    )
    def k(Wt_hbm, tail_hbm, out_hbm, wts, wrs, isems, osems):
        wid = lax.axis_index("s") * _NC + lax.axis_index("c")
        iota = lax.iota(jnp.int32, 16)
        c2base = (iota & 3) * _D

        def fetch(blk, b):
            pltpu.async_copy(
                Wt_hbm.at[:, pl.ds(blk * 256, 256)], wts[b], isems[b])

        def wait_fetch(b):
            pltpu.make_async_copy(
                Wt_hbm.at[:, pl.ds(0, 256)], wts[b], isems[b]).wait()

        def put(blk, b):
            pltpu.async_copy(
                wrs[b], out_hbm.at[pl.ds(blk * 64, 64), :], osems[b])

        def wait_put(b):
            pltpu.make_async_copy(
                wrs[b], out_hbm.at[pl.ds(0, 64), :], osems[b]).wait()

        for b in range(2):
            @pl.when(b * _NW + wid < _NBLK)
            def _():
                fetch(b * _NW + wid, b)

        def body(j, carry):
            for b in range(2):
                i = 2 * j + b
                blk = i * _NW + wid

                @pl.when(blk < _NBLK)
                def _():
                    wait_fetch(b)

                    @pl.when(j >= 1)
                    def _():
                        wait_put(b)

                    @plsc.parallel_loop(0, 16, unroll=8)
                    def _(cg):
                        r2 = cg * 4 + (iota >> 2)
                        vals = [wts[b][e, pl.ds(cg * 16, 16)]
                                for e in range(_D)]
                        for e in range(_D):
                            plsc.store_scatter(
                                wrs[b], [r2, c2base + e], vals[e])

                    put(blk, b)
                    nxt = blk + 2 * _NW

                    @pl.when(nxt < _NBLK)
                    def _():
                        fetch(nxt, b)
            return carry

        lax.fori_loop(0, (_BPW_A + 1) // 2, body, 0, unroll=False)
        for b in range(2):
            wait_put(b)

        @pl.when(wid == _NW - 1)
        def _():
            pltpu.sync_copy(tail_hbm, wts[0].at[pl.ds(0, 16), pl.ds(0, 128)])
            pltpu.sync_copy(wts[0].at[pl.ds(0, 16), pl.ds(0, 128)],
                            out_hbm.at[pl.ds(_V // 4 - 16, 16), :])

    return k(Wt, tail16)


@jax.jit
def _sc_embed(xT, table2):
    mesh = plsc.VectorSubcoreMesh(core_axis_name="c", subcore_axis_name="s")

    @functools.partial(
        pl.kernel,
        mesh=mesh,
        out_type=jax.ShapeDtypeStruct((_H, _D, _B), jnp.float32),
        scratch_types=[
            pltpu.VMEM((_HB, _BB), jnp.int32),
            [pltpu.VMEM((_C,), jnp.int32) for _ in range(2)],
            [pltpu.VMEM((_C,), jnp.int32) for _ in range(2)],
            [pltpu.VMEM((_C, 128), jnp.float32) for _ in range(2)],
            [pltpu.VMEM((_C // 128, 8, _BB), jnp.float32) for _ in range(8)],
            [pltpu.SemaphoreType.DMA for _ in range(2)],
            [pltpu.SemaphoreType.DMA for _ in range(2)],
        ],
        compiler_params=pltpu.CompilerParams(needs_layout_passes=False),
    )
    def k(xT_hbm, tab_hbm, out_hbm, xt_v, q_v, a_v, rows, ots, gsems, ssems):
        wid = lax.axis_index("s") * _NC + lax.axis_index("c")
        iota = lax.iota(jnp.int32, 16)

        def drain_stores(b):
            for _ in range(8):
                pltpu.make_async_copy(
                    ots[0].at[0],
                    out_hbm.at[0, pl.ds(0, 8), pl.ds(0, _BB)],
                    ssems[b]).wait()

        def unit_body(uu, carry):
            u = uu * _NW + wid
            b0 = (u % _NBB) * _BB
            h0 = (u // _NBB) * _HB
            pltpu.sync_copy(
                xT_hbm.at[pl.ds(h0, _HB), pl.ds(b0, _BB)], xt_v)

            def build_qa(qq, b):
                for g in range(_C // 16):
                    v = xt_v[qq * 2 + g // 8, pl.ds((g % 8) * 16, 16)]
                    q_v[b][pl.ds(g * 16, 16)] = v >> 2
                    a_v[b][pl.ds(g * 16, 16)] = v & 3

            def gather_q(b):
                return pltpu.async_copy(
                    tab_hbm.at[q_v[b]], rows[b], gsems[b])

            build_qa(0, 0)
            build_qa(1, 1)
            gathers = [gather_q(0), gather_q(1)]
            for qq in range(_NQ):
                b = qq % 2
                gathers[qq].wait()
                if qq < 2:
                    @pl.when(uu != 0)
                    def _():
                        drain_stores(b)
                else:
                    drain_stores(b)

                @plsc.parallel_loop(0, _C // 16, unroll=8)
                def _(j16):
                    ridx = j16 * 16 + iota
                    a16 = a_v[b][pl.ds(j16 * 16, 16)]
                    cbase = a16 * _D
                    hh = j16 >> 3
                    cg16 = (j16 & 7) * 16
                    vals = [plsc.load_gather(rows[b], [ridx, cbase + e])
                            for e in range(_D)]
                    for e in range(_D):
                        ots[b * 4 + (e >> 3)][hh, e & 7,
                                              pl.ds(cg16, 16)] = vals[e]

                for hh in range(2):
                    for t0 in range(4):
                        pltpu.async_copy(
                            ots[b * 4 + t0].at[hh],
                            out_hbm.at[h0 + qq * 2 + hh, pl.ds(t0 * 8, 8),
                                       pl.ds(b0, _BB)],
                            ssems[b])
                nxt = qq + 2
                if nxt < _NQ:
                    build_qa(nxt, b)
                    gathers.append(gather_q(b))
            return carry

        lax.fori_loop(0, _UPW, unit_body, 0, unroll=False)
        for b in range(2):
            drain_stores(b)

    return k(xT, table2)


def kernel(x, W):
    xT = x.T.astype(jnp.int32)
    tail16 = lax.slice(W, (_V - 64, 0), (_V, _D)).reshape(16, 128)
    table2 = _sc_transpose(W.T, tail16)
    outT = _sc_embed(xT, table2)
    return outT.transpose(2, 0, 1)
